# baseline (speedup 1.0000x reference)
import functools

import jax
import jax.numpy as jnp
from jax import lax
from jax.experimental import pallas as pl
from jax.experimental.pallas import tpu as pltpu
from jax.experimental.pallas import tpu_sc as plsc


def _sc_pad_transpose(tableT, tail, num_cores, num_subcores):
    D, V = tableT.shape
    NV = D // 16
    NW = num_cores * num_subcores
    W = 256
    CPW = (V // (NW * W))
    REM_BASE = NW * CPW * W
    REM_EACH = 128
    REM_WORKERS = (V - REM_BASE) // REM_EACH
    TAIL = tail.shape[0]

    mesh = plsc.VectorSubcoreMesh(core_axis_name="c", subcore_axis_name="s")
    NBUF = 2

    @functools.partial(
        pl.kernel,
        mesh=mesh,
        out_type=jax.ShapeDtypeStruct((V, 128), jnp.float32),
        scratch_types=[
            pltpu.VMEM((NBUF, D, W), jnp.float32),
            pltpu.VMEM((NBUF, W, 128), jnp.float32),
            pltpu.VMEM((TAIL, D), jnp.float32),
            [pltpu.SemaphoreType.DMA] * NBUF,
            [pltpu.SemaphoreType.DMA] * NBUF,
        ],
        compiler_params=pltpu.CompilerParams(needs_layout_passes=False),
    )
    def padt(tt_hbm, tail_hbm, out_hbm, in_v, out_v, tail_v, isems, osems):
        wid = lax.axis_index("s") * num_cores + lax.axis_index("c")
        base = wid * CPW * W
        iota = lax.iota(jnp.int32, 16)
        rows16 = [jnp.int32(16 * v) + iota for v in range(NV)]

        def issue_in(c, slot):
            pltpu.make_async_copy(
                tt_hbm.at[:, pl.ds(base + c * W, W)],
                in_v.at[slot],
                isems[slot],
            ).start()

        def wait_in(slot):
            pltpu.make_async_copy(
                tt_hbm.at[:, pl.ds(0, W)], in_v.at[slot], isems[slot]
            ).wait()

        def transpose_rows(slot, n):
            def row_body(i, carry2):
                isp = jnp.full((16,), i, jnp.int32)
                for v in range(NV):
                    out_v[slot, i, pl.ds(16 * v, 16)] = plsc.load_gather(
                        in_v.at[slot], [rows16[v], isp]
                    )
                return carry2

            lax.fori_loop(0, n, row_body, 0, unroll=4)

        def wait_out(slot):
            pltpu.make_async_copy(
                out_hbm.at[pl.ds(0, W)],
                out_v.at[slot],
                osems[slot],
            ).wait()

        for slot in range(NBUF):
            issue_in(slot, slot)

        def chunk_body(c, carry):
            for slot in range(NBUF):
                cc = c * NBUF + slot
                wait_in(slot)
                transpose_rows(slot, W)

                @pl.when(cc >= NBUF)
                def _():
                    wait_out(slot)

                pltpu.make_async_copy(
                    out_v.at[slot],
                    out_hbm.at[pl.ds(base + cc * W, W)],
                    osems[slot],
                ).start()

                nc = cc + NBUF

                @pl.when(nc < CPW)
                def _():
                    issue_in(nc, slot)

            return carry

        lax.fori_loop(0, CPW // NBUF, chunk_body, 0)
        for slot in range(NBUF):
            wait_out(slot)

        @pl.when(wid < REM_WORKERS)
        def _():
            rb = REM_BASE + wid * REM_EACH
            pltpu.sync_copy(
                tt_hbm.at[:, pl.ds(rb, REM_EACH)],
                in_v.at[0, pl.ds(0, D), pl.ds(0, REM_EACH)],
            )
            transpose_rows(0, REM_EACH)
            pltpu.sync_copy(
                out_v.at[0, pl.ds(0, REM_EACH)],
                out_hbm.at[pl.ds(rb, REM_EACH)],
            )

        @pl.when(wid == REM_WORKERS)
        def _():
            pltpu.sync_copy(tail_hbm, tail_v)

            def tail_body(i, carry2):
                for v in range(NV):
                    out_v[0, i, pl.ds(16 * v, 16)] = tail_v[
                        i, pl.ds(16 * v, 16)
                    ]
                return carry2

            lax.fori_loop(0, TAIL, tail_body, 0, unroll=4)
            pltpu.sync_copy(
                out_v.at[0, pl.ds(0, TAIL)],
                out_hbm.at[pl.ds(V - TAIL, TAIL)],
            )

    return padt(tableT, tail)


def _sc_pool_sum(idx, table, num_cores, num_subcores):
    B, R = idx.shape
    V, DP = table.shape
    D = 64
    NV = D // 16
    NI = R // 16
    BPW = B // (num_cores * num_subcores)

    mesh = plsc.VectorSubcoreMesh(core_axis_name="c", subcore_axis_name="s")
    NBUF = 2

    @functools.partial(
        pl.kernel,
        mesh=mesh,
        out_type=jax.ShapeDtypeStruct((B, D), jnp.float32),
        scratch_types=[
            pltpu.VMEM((BPW, R), jnp.int32),
            pltpu.VMEM((NBUF, R, DP), jnp.float32),
            pltpu.VMEM((BPW, D), jnp.float32),
            [pltpu.SemaphoreType.DMA] * NBUF,
        ],
    )
    def pool(idx_hbm, table_hbm, out_hbm, idx_v, rows_v, acc_v, sems):
        wid = lax.axis_index("s") * num_cores + lax.axis_index("c")
        base = wid * BPW
        pltpu.sync_copy(idx_hbm.at[pl.ds(base, BPW)], idx_v)

        def issue(b, slot):
            for k in range(NI):
                ivec = idx_v[b, pl.ds(16 * k, 16)]
                pltpu.make_async_copy(
                    table_hbm.at[ivec],
                    rows_v.at[slot, pl.ds(16 * k, 16)],
                    sems[slot],
                ).start()

        def wait_slot(slot):
            pltpu.make_async_copy(
                table_hbm.at[pl.ds(0, R)],
                rows_v.at[slot],
                sems[slot],
            ).wait()

        for slot in range(NBUF):
            issue(slot, slot)

        zero = jnp.zeros((16,), jnp.float32)

        def group_body(g, carry):
            for slot in range(NBUF):
                b = g * NBUF + slot
                wait_slot(slot)

                def acc_body(r, acc):
                    return tuple(
                        acc[v] + rows_v[slot, r, pl.ds(16 * v, 16)]
                        for v in range(NV)
                    )

                acc = lax.fori_loop(0, R, acc_body, (zero,) * NV, unroll=8)
                for v in range(NV):
                    acc_v[b, pl.ds(16 * v, 16)] = acc[v]

                nb = b + NBUF

                @pl.when(nb < BPW)
                def _():
                    issue(nb, slot)

            return carry

        lax.fori_loop(0, BPW // NBUF, group_body, 0)
        pltpu.sync_copy(acc_v, out_hbm.at[pl.ds(base, BPW)])

    return pool(idx, table)


def _mask_correct_mlp(inputs, sums, table0, W1, b1, W2, b2, pad_per_row):
    B, L = inputs.shape
    D = sums.shape[1]
    H = W1.shape[1]
    BLK = 1024

    def body(inp_ref, sums_ref, t0_ref, W1_ref, b1_ref, W2_ref, b2_ref, out_ref):
        cnt = jnp.sum(
            (inp_ref[...] == 0).astype(jnp.float32), axis=1, keepdims=True
        )
        pooled = sums_ref[...] - (cnt + pad_per_row) * t0_ref[...]
        h = jnp.tanh(
            jnp.dot(pooled, W1_ref[...], preferred_element_type=jnp.float32)
            + b1_ref[...]
        )
        out_ref[...] = (
            jnp.dot(h, W2_ref[...], preferred_element_type=jnp.float32)
            + b2_ref[...]
        )

    return pl.pallas_call(
        body,
        grid=(B // BLK,),
        in_specs=[
            pl.BlockSpec((BLK, L), lambda i: (i, 0)),
            pl.BlockSpec((BLK, D), lambda i: (i, 0)),
            pl.BlockSpec((1, D), lambda i: (0, 0)),
            pl.BlockSpec((D, H), lambda i: (0, 0)),
            pl.BlockSpec((1, H), lambda i: (0, 0)),
            pl.BlockSpec((H, D), lambda i: (0, 0)),
            pl.BlockSpec((1, D), lambda i: (0, 0)),
        ],
        out_specs=pl.BlockSpec((BLK, D), lambda i: (i, 0)),
        out_shape=jax.ShapeDtypeStruct((B, D), jnp.float32),
    )(inputs, sums, table0, W1, b1, W2, b2)


def kernel(inputs, table, W1, b1, W2, b2):
    B, L = inputs.shape
    info = plsc.get_sparse_core_info()

    R = -(-L // 16) * 16
    pad = R - L
    idx = jnp.pad(inputs, ((0, 0), (0, pad)))

    tail_rows = table.shape[0] % 128
    table_p = _sc_pad_transpose(
        table.T,
        table[table.shape[0] - tail_rows :],
        info.num_cores,
        info.num_subcores,
    )

    sums = _sc_pool_sum(idx, table_p, info.num_cores, info.num_subcores)
    return _mask_correct_mlp(
        inputs,
        sums,
        table[0:1],
        W1,
        b1.reshape(1, -1),
        W2,
        b2.reshape(1, -1),
        float(pad),
    )


# speedup vs baseline: 1.4383x; 1.4383x over previous
import functools

import jax
import jax.numpy as jnp
from jax import lax
from jax.experimental import pallas as pl
from jax.experimental.pallas import tpu as pltpu
from jax.experimental.pallas import tpu_sc as plsc


def _sc_pool_sum(idx, table2, num_cores, num_subcores):
    B, R = idx.shape
    _, DP = table2.shape
    D = DP // 2
    NV = D // 16
    NI = R // 16
    BPW = B // (num_cores * num_subcores)

    mesh = plsc.VectorSubcoreMesh(core_axis_name="c", subcore_axis_name="s")
    NBUF = 2

    @functools.partial(
        pl.kernel,
        mesh=mesh,
        out_type=jax.ShapeDtypeStruct((B, D), jnp.float32),
        scratch_types=[
            pltpu.VMEM((BPW, R), jnp.int32),
            pltpu.VMEM((NBUF, R, DP), jnp.float32),
            pltpu.VMEM((BPW, D), jnp.float32),
            [pltpu.SemaphoreType.DMA] * NBUF,
        ],
        compiler_params=pltpu.CompilerParams(needs_layout_passes=False),
    )
    def pool(idx_hbm, table_hbm, out_hbm, idx_v, rows_v, acc_v, sems):
        wid = lax.axis_index("s") * num_cores + lax.axis_index("c")
        base = wid * BPW
        pltpu.sync_copy(idx_hbm.at[pl.ds(base, BPW)], idx_v)

        def issue(b, slot):
            for k in range(NI):
                ivec = idx_v[b, pl.ds(16 * k, 16)] >> 1
                pltpu.make_async_copy(
                    table_hbm.at[ivec],
                    rows_v.at[slot, pl.ds(16 * k, 16)],
                    sems[slot],
                ).start()

        def wait_slot(slot):
            pltpu.make_async_copy(
                table_hbm.at[pl.ds(0, R)], rows_v.at[slot], sems[slot]
            ).wait()

        for slot in range(NBUF):
            issue(slot, slot)

        zero = jnp.zeros((16,), jnp.float32)
        iota = lax.iota(jnp.int32, 16)
        cols = [jnp.int32(16 * v) + iota for v in range(NV)]

        def group_body(g, carry):
            for slot in range(NBUF):
                b = g * NBUF + slot
                wait_slot(slot)
                bsp = jnp.full((16,), b, jnp.int32)
                rows_s = rows_v.at[slot]

                def acc_body(r, acc):
                    rsp = jnp.full((16,), r, jnp.int32)
                    off = (plsc.load_gather(idx_v, [bsp, rsp]) & 1) * D
                    return tuple(
                        acc[v]
                        + plsc.load_gather(rows_s, [rsp, off + cols[v]])
                        for v in range(NV)
                    )

                acc = lax.fori_loop(0, R, acc_body, (zero,) * NV, unroll=4)
                for v in range(NV):
                    acc_v[b, pl.ds(16 * v, 16)] = acc[v]

                nb = b + NBUF

                @pl.when(nb < BPW)
                def _():
                    issue(nb, slot)

            return carry

        lax.fori_loop(0, BPW // NBUF, group_body, 0)
        pltpu.sync_copy(acc_v, out_hbm.at[pl.ds(base, BPW)])

    return pool(idx, table2)


def _mask_correct_mlp(inputs, sums, table0, W1, b1, W2, b2, pad_per_row):
    B, L = inputs.shape
    D = sums.shape[1]
    H = W1.shape[1]
    BLK = 1024

    def body(inp_ref, sums_ref, t0_ref, W1_ref, b1_ref, W2_ref, b2_ref, out_ref):
        cnt = jnp.sum(
            (inp_ref[...] == 0).astype(jnp.float32), axis=1, keepdims=True
        )
        pooled = sums_ref[...] - (cnt + pad_per_row) * t0_ref[...]
        h = jnp.tanh(
            jnp.dot(pooled, W1_ref[...], preferred_element_type=jnp.float32)
            + b1_ref[...]
        )
        out_ref[...] = (
            jnp.dot(h, W2_ref[...], preferred_element_type=jnp.float32)
            + b2_ref[...]
        )

    return pl.pallas_call(
        body,
        grid=(B // BLK,),
        in_specs=[
            pl.BlockSpec((BLK, L), lambda i: (i, 0)),
            pl.BlockSpec((BLK, D), lambda i: (i, 0)),
            pl.BlockSpec((1, D), lambda i: (0, 0)),
            pl.BlockSpec((D, H), lambda i: (0, 0)),
            pl.BlockSpec((1, H), lambda i: (0, 0)),
            pl.BlockSpec((H, D), lambda i: (0, 0)),
            pl.BlockSpec((1, D), lambda i: (0, 0)),
        ],
        out_specs=pl.BlockSpec((BLK, D), lambda i: (i, 0)),
        out_shape=jax.ShapeDtypeStruct((B, D), jnp.float32),
    )(inputs, sums, table0, W1, b1, W2, b2)


def kernel(inputs, table, W1, b1, W2, b2):
    B, L = inputs.shape

    info = plsc.get_sparse_core_info()

    R = -(-L // 16) * 16
    pad = R - L
    idx = jnp.pad(inputs, ((0, 0), (0, pad)))

    sums = _sc_pool_sum(
        idx, table.reshape(-1, 2 * table.shape[1]),
        info.num_cores, info.num_subcores,
    )
    return _mask_correct_mlp(
        inputs,
        sums,
        table[0:1],
        W1,
        b1.reshape(1, -1),
        W2,
        b2.reshape(1, -1),
        float(pad),
    )
